# Initial kernel scaffold; baseline (speedup 1.0000x reference)
import jax
import jax.numpy as jnp
from jax.experimental import pallas as pl


def kernel(input_ids, loras_a, loras_b, lora_indices, emb, W_lin, b_lin, W_head, b_head):
    raise NotImplementedError("write your pallas kernel here")


# speedup vs baseline: 18.1026x; 18.1026x over previous
import functools

import jax
import jax.numpy as jnp
from jax import lax
from jax.experimental import pallas as pl
from jax.experimental.pallas import tpu as pltpu
from jax.experimental.pallas import tpu_sc as plsc

H = 10
R = 2
NUM_LORAS = 64
DPAD = 16
NC, NS = 2, 16
NW = NC * NS
IDX_CHUNK = 128


def _table_body(x_ref, a0_ref, a1_ref, b0_ref, b1_ref, wl_ref, bl_ref,
                wh_ref, bh_ref, out_ref):
    x = x_ref[...]
    base = jnp.dot(x, wl_ref[...], preferred_element_type=jnp.float32) + bl_ref[...]
    xa0 = jnp.sum(x * a0_ref[...], axis=1, keepdims=True)
    xa1 = jnp.sum(x * a1_ref[...], axis=1, keepdims=True)
    lora = xa0 * b0_ref[...] + xa1 * b1_ref[...]
    y = base + lora
    out_ref[...] = jnp.dot(y, wh_ref[...], preferred_element_type=jnp.float32) + bh_ref[...]


def _build_table(emb, loras_a, loras_b, W_lin, b_lin, W_head, b_head):
    x640 = jnp.repeat(emb, NUM_LORAS, axis=0)
    a0 = jnp.tile(loras_a[:, :, 0], (H, 1))
    a1 = jnp.tile(loras_a[:, :, 1], (H, 1))
    b0 = jnp.tile(loras_b[:, 0, :], (H, 1))
    b1 = jnp.tile(loras_b[:, 1, :], (H, 1))
    wl = W_lin.T
    bl = b_lin.reshape(1, H)
    wh = jnp.pad(W_head.T, ((0, 0), (0, DPAD - H)))
    bh = jnp.pad(b_head, (0, DPAD - H)).reshape(1, DPAD)
    return pl.pallas_call(
        _table_body,
        out_shape=jax.ShapeDtypeStruct((H * NUM_LORAS, DPAD), jnp.float32),
    )(x640, a0, a1, b0, b1, wl, bl, wh, bh)


def _gather_call(table, ids, lor):
    B = ids.shape[0]
    b_per_w = B // NW
    n_chunks = b_per_w // IDX_CHUNK
    mesh = plsc.VectorSubcoreMesh(core_axis_name="c", subcore_axis_name="s",
                                  num_cores=NC, num_subcores=NS)

    @functools.partial(
        pl.kernel,
        out_type=jax.ShapeDtypeStruct((B, DPAD), jnp.float32),
        mesh=mesh,
        compiler_params=pltpu.CompilerParams(use_tc_tiling_on_sc=False),
        scratch_types=[
            pltpu.VMEM((b_per_w,), jnp.int32),
            pltpu.VMEM((b_per_w,), jnp.int32),
            pltpu.VMEM((b_per_w,), jnp.int32),
            pltpu.VMEM((b_per_w, DPAD), jnp.float32),
            pltpu.SemaphoreType.DMA,
        ],
    )
    def sc_gather(table_hbm, ids_hbm, lor_hbm, out_hbm,
                  ids_v, lor_v, idx_v, rows_v, sem):
        wid = lax.axis_index("s") * NC + lax.axis_index("c")
        base = wid * b_per_w
        pltpu.sync_copy(ids_hbm.at[pl.ds(base, b_per_w)], ids_v)
        pltpu.sync_copy(lor_hbm.at[pl.ds(base, b_per_w)], lor_v)

        def fuse(i, carry):
            s = pl.ds(i * 16, 16)
            idx_v[s] = ids_v[s] * NUM_LORAS + lor_v[s]
            return carry
        lax.fori_loop(0, b_per_w // 16, fuse, 0)

        copies = []
        for j in range(n_chunks):
            s = pl.ds(j * IDX_CHUNK, IDX_CHUNK)
            copies.append(
                pltpu.async_copy(table_hbm.at[idx_v.at[s]], rows_v.at[s], sem))
        for c in copies:
            c.wait()
        pltpu.sync_copy(rows_v, out_hbm.at[pl.ds(base, b_per_w)])

    return sc_gather(table, ids, lor)


def kernel(input_ids, loras_a, loras_b, lora_indices, emb, W_lin, b_lin,
           W_head, b_head):
    table = _build_table(emb, loras_a, loras_b, W_lin, b_lin, W_head, b_head)
    ids = input_ids.astype(jnp.int32)
    lor = lora_indices.astype(jnp.int32)
    out = _gather_call(table, ids, lor)
    return out[:, :H]
